# baseline (speedup 1.0000x reference)
import functools

import jax
import jax.numpy as jnp
from jax import lax
from jax.experimental import pallas as pl
from jax.experimental.pallas import tpu as pltpu
from jax.experimental.pallas import tpu_sc as plsc

_BATCH = 16384
_NCLS = 1000
_LANES = 16
_NCORES = 2
_NSUB = 16
_NTILES = _NCORES * _NSUB
_ROWS_PER_TILE = _BATCH // _NTILES
_CHUNK = 32
_NCHUNK = _ROWS_PER_TILE // _CHUNK
_GROUPS = _CHUNK // _LANES
_BUFW = _CHUNK * _NCLS


def _onehot_body(labels_hbm, out_hbm, labels_v, buf0, buf1, sem0, sem1):
    wid = lax.axis_index("s") * _NCORES + lax.axis_index("c")
    base = wid * _ROWS_PER_TILE

    pltpu.sync_copy(labels_hbm.at[pl.ds(base, _ROWS_PER_TILE)], labels_v)

    bufs = (buf0, buf1)
    sems = (sem0, sem1)

    zeros16 = jnp.zeros((_LANES,), jnp.float32)
    ones16 = jnp.ones((_LANES,), jnp.float32)
    row_base = lax.iota(jnp.int32, _LANES) * _NCLS

    _UNROLL = 8
    def _zero_step(i, carry):
        off = i * (_LANES * _UNROLL)
        for b in bufs:
            for j in range(_UNROLL):
                b[pl.ds(off + j * _LANES, _LANES)] = zeros16
        return carry

    lax.fori_loop(0, _BUFW // (_LANES * _UNROLL), _zero_step, 0)

    copies = [None, None]
    prev_idx = [None, None]
    for c in range(_NCHUNK):
        b = c % 2
        if copies[b] is not None:
            copies[b].wait()
            for g in range(_GROUPS):
                plsc.store_scatter(bufs[b], [prev_idx[b][g]], zeros16)
        idxs = []
        for g in range(_GROUPS):
            col = labels_v[pl.ds(c * _CHUNK + g * _LANES, _LANES)]
            flat = row_base + (g * _LANES * _NCLS) + col
            plsc.store_scatter(bufs[b], [flat], ones16)
            idxs.append(flat)
        prev_idx[b] = idxs
        copies[b] = pltpu.async_copy(
            bufs[b],
            out_hbm.at[pl.ds((base + c * _CHUNK) * _NCLS, _BUFW)],
            sems[b],
        )
    copies[0].wait()
    copies[1].wait()


_onehot = functools.partial(
    pl.kernel,
    out_type=jax.ShapeDtypeStruct((_BATCH * _NCLS,), jnp.float32),
    mesh=plsc.VectorSubcoreMesh(core_axis_name="c", subcore_axis_name="s"),
    compiler_params=pltpu.CompilerParams(needs_layout_passes=False),
    scratch_types=[
        pltpu.VMEM((_ROWS_PER_TILE,), jnp.int32),
        pltpu.VMEM((_BUFW,), jnp.float32),
        pltpu.VMEM((_BUFW,), jnp.float32),
        pltpu.SemaphoreType.DMA,
        pltpu.SemaphoreType.DMA,
    ],
)(_onehot_body)


def kernel(labels):
    flat = _onehot(labels.astype(jnp.int32))
    return flat.reshape(_BATCH, _NCLS)


# speedup vs baseline: 1.6150x; 1.6150x over previous
import functools

import jax
import jax.numpy as jnp
from jax import lax
from jax.experimental import pallas as pl
from jax.experimental.pallas import tpu as pltpu
from jax.experimental.pallas import tpu_sc as plsc

_BATCH = 16384
_NCLS = 1000
_LANES = 16
_NCORES = 2
_NSUB = 16
_NTILES = _NCORES * _NSUB
_ROWS_PER_TILE = _BATCH // _NTILES
_CHUNK = 32
_NCHUNK = _ROWS_PER_TILE // _CHUNK
_GROUPS = _CHUNK // _LANES


def _onehot_body(labels_hbm, out_hbm, labels_v, buf0, buf1, sem0, sem1):
    wid = lax.axis_index("s") * _NCORES + lax.axis_index("c")
    base = wid * _ROWS_PER_TILE

    pltpu.sync_copy(labels_hbm.at[pl.ds(base, _ROWS_PER_TILE)], labels_v)

    bufs = (buf0, buf1)
    sems = (sem0, sem1)

    zeros16 = jnp.zeros((_LANES,), jnp.float32)
    ones16 = jnp.ones((_LANES,), jnp.float32)
    iota16 = lax.iota(jnp.int32, _LANES)

    def _zero_row(r, carry):
        for b in bufs:
            for off in range(0, _NCLS - _LANES, _LANES):
                b[r, pl.ds(off, _LANES)] = zeros16
            b[r, pl.ds(_NCLS - _LANES, _LANES)] = zeros16
        return carry

    lax.fori_loop(0, _CHUNK, _zero_row, 0)

    copies = [None, None]
    prev_cols = [None, None]
    for c in range(_NCHUNK):
        b = c % 2
        if copies[b] is not None:
            copies[b].wait()
            for g in range(_GROUPS):
                plsc.store_scatter(
                    bufs[b], [iota16 + g * _LANES, prev_cols[b][g]], zeros16
                )
        cols = []
        for g in range(_GROUPS):
            col = labels_v[pl.ds(c * _CHUNK + g * _LANES, _LANES)]
            plsc.store_scatter(bufs[b], [iota16 + g * _LANES, col], ones16)
            cols.append(col)
        prev_cols[b] = cols
        copies[b] = pltpu.async_copy(
            bufs[b], out_hbm.at[pl.ds(base + c * _CHUNK, _CHUNK)], sems[b]
        )
    copies[0].wait()
    copies[1].wait()


_onehot = functools.partial(
    pl.kernel,
    out_type=jax.ShapeDtypeStruct((_BATCH, _NCLS), jnp.float32),
    mesh=plsc.VectorSubcoreMesh(core_axis_name="c", subcore_axis_name="s"),
    compiler_params=pltpu.CompilerParams(
        needs_layout_passes=False, use_tc_tiling_on_sc=True
    ),
    scratch_types=[
        pltpu.VMEM((_ROWS_PER_TILE,), jnp.int32),
        pltpu.VMEM((_CHUNK, _NCLS), jnp.float32),
        pltpu.VMEM((_CHUNK, _NCLS), jnp.float32),
        pltpu.SemaphoreType.DMA,
        pltpu.SemaphoreType.DMA,
    ],
)(_onehot_body)


def kernel(labels):
    return _onehot(labels.astype(jnp.int32))


# speedup vs baseline: 4.0520x; 2.5090x over previous
import functools

import jax
import jax.numpy as jnp
from jax import lax
from jax.experimental import pallas as pl
from jax.experimental.pallas import tpu as pltpu
from jax.experimental.pallas import tpu_sc as plsc

_BATCH = 16384
_NCLS = 1000
_LANES = 16
_NCORES = 2
_NSUB = 16
_NTILES = _NCORES * _NSUB
_COLS_PER_TILE = _BATCH // _NTILES
_NGRP = _COLS_PER_TILE // _LANES
_CCHUNK = 40
_NCHUNK = _NCLS // _CCHUNK
_UNROLL = 4


def _onehot_body(labels_hbm, out_hbm, labels_v, buf0, buf1, sem0, sem1, seml):
    wid = lax.axis_index("s") * _NCORES + lax.axis_index("c")
    base = wid * _COLS_PER_TILE

    lbl_copy = pltpu.async_copy(
        labels_hbm.at[pl.ds(base, _COLS_PER_TILE)], labels_v, seml
    )

    zeros16 = jnp.zeros((_LANES,), jnp.float32)
    ones16 = jnp.ones((_LANES,), jnp.float32)
    iota16 = lax.iota(jnp.int32, _LANES)

    def _zero_buf(buf):
        def _row(r, carry):
            for off in range(0, _COLS_PER_TILE, _LANES):
                buf[r, pl.ds(off, _LANES)] = zeros16
            return carry

        lax.fori_loop(0, _CCHUNK, _row, 0)

    def _scatter_pass(buf, c0_set, c0_restore):
        def _step(i, carry):
            for j in range(_UNROLL):
                g = i * _UNROLL + j
                col = g * _LANES + iota16
                lbl = labels_v[pl.ds(g * _LANES, _LANES)]
                if c0_restore is not None:
                    rel = lbl - c0_restore
                    m = (rel >= 0) & (rel < _CCHUNK)
                    plsc.store_scatter(buf, [rel, col], zeros16, mask=m)
                rel = lbl - c0_set
                m = (rel >= 0) & (rel < _CCHUNK)
                plsc.store_scatter(buf, [rel, col], ones16, mask=m)
            return carry

        lax.fori_loop(0, _NGRP // _UNROLL, _step, 0)

    def _dma(buf, c0, sem):
        return pltpu.async_copy(
            buf,
            out_hbm.at[pl.ds(c0, _CCHUNK), pl.ds(base, _COLS_PER_TILE)],
            sem,
        )

    _zero_buf(buf0)
    lbl_copy.wait()
    _scatter_pass(buf0, 0, None)
    _dma(buf0, 0, sem0)
    _zero_buf(buf1)
    _scatter_pass(buf1, _CCHUNK, None)
    _dma(buf1, _CCHUNK, sem1)

    def _pair(p, carry):
        c0 = 2 * p * _CCHUNK
        pltpu.make_async_copy(
            buf0,
            out_hbm.at[pl.ds(0, _CCHUNK), pl.ds(base, _COLS_PER_TILE)],
            sem0,
        ).wait()
        _scatter_pass(buf0, c0, c0 - 2 * _CCHUNK)
        _dma(buf0, c0, sem0)
        pltpu.make_async_copy(
            buf1,
            out_hbm.at[pl.ds(0, _CCHUNK), pl.ds(base, _COLS_PER_TILE)],
            sem1,
        ).wait()
        _scatter_pass(buf1, c0 + _CCHUNK, c0 - _CCHUNK)
        _dma(buf1, c0 + _CCHUNK, sem1)
        return carry

    lax.fori_loop(1, (_NCHUNK - 1) // 2, _pair, 0)

    c_last = (_NCHUNK - 1) * _CCHUNK
    pltpu.make_async_copy(
        buf0,
        out_hbm.at[pl.ds(0, _CCHUNK), pl.ds(base, _COLS_PER_TILE)],
        sem0,
    ).wait()
    _scatter_pass(buf0, c_last, c_last - 2 * _CCHUNK)
    _dma(buf0, c_last, sem0).wait()
    pltpu.make_async_copy(
        buf1,
        out_hbm.at[pl.ds(0, _CCHUNK), pl.ds(base, _COLS_PER_TILE)],
        sem1,
    ).wait()


_onehot_t = functools.partial(
    pl.kernel,
    out_type=jax.ShapeDtypeStruct((_NCLS, _BATCH), jnp.float32),
    mesh=plsc.VectorSubcoreMesh(core_axis_name="c", subcore_axis_name="s"),
    compiler_params=pltpu.CompilerParams(
        needs_layout_passes=False, use_tc_tiling_on_sc=True
    ),
    scratch_types=[
        pltpu.VMEM((_COLS_PER_TILE,), jnp.int32),
        pltpu.VMEM((_CCHUNK, _COLS_PER_TILE), jnp.float32),
        pltpu.VMEM((_CCHUNK, _COLS_PER_TILE), jnp.float32),
        pltpu.SemaphoreType.DMA,
        pltpu.SemaphoreType.DMA,
        pltpu.SemaphoreType.DMA,
    ],
)(_onehot_body)


def kernel(labels):
    return _onehot_t(labels.astype(jnp.int32)).T
